# baseline (speedup 1.0000x reference)
import functools

import jax
import jax.numpy as jnp
from jax import lax
from jax.experimental import pallas as pl
from jax.experimental.pallas import tpu as pltpu
from jax.experimental.pallas import tpu_sc as plsc

_NC = 2
_NS = 16
_NW = _NC * _NS
_L = 16


def _softmax_body(alpha_ref, out_ref):
    a = alpha_ref[...]
    m = jnp.max(a)
    e = jnp.exp(a - m)
    out_ref[...] = e / jnp.sum(e)


def _softmax_table(alpha_padded_2d):
    return pl.pallas_call(
        _softmax_body,
        out_shape=jax.ShapeDtypeStruct(alpha_padded_2d.shape, jnp.float32),
    )(alpha_padded_2d)


def _sc_gather_body(nchunks, chunk, b_per_w, t_per_t, table_hbm, idx_hbm,
                    out_hbm, idx_v0, idx_v1, rows_v0, rows_v1, shared,
                    sem_i0, sem_i1, sem_g0, sem_g1, sem_o0, sem_o1):
    sid = lax.axis_index("s")
    wid = sid * _NC + lax.axis_index("c")
    base = wid * b_per_w
    idx_v = [idx_v0, idx_v1]
    rows_v = [rows_v0, rows_v1]
    sem_i = [sem_i0, sem_i1]
    sem_g = [sem_g0, sem_g1]
    sem_o = [sem_o0, sem_o1]

    cp_i = [None] * nchunks
    cp_g = [None, None]
    cp_o = [None, None]
    cp_i[0] = pltpu.async_copy(idx_hbm.at[pl.ds(base, chunk)], idx_v0, sem_i0)
    if nchunks > 1:
        cp_i[1] = pltpu.async_copy(idx_hbm.at[pl.ds(base + chunk, chunk)],
                                   idx_v1, sem_i1)

    toff = sid * t_per_t
    done = 0
    while done < t_per_t:
        piece = min(chunk, t_per_t - done)
        pltpu.sync_copy(table_hbm.at[pl.ds(toff + done, piece)],
                        rows_v0.at[pl.ds(0, piece)])
        pltpu.sync_copy(rows_v0.at[pl.ds(0, piece)],
                        shared.at[pl.ds(toff + done, piece)])
        done += piece
    plsc.subcore_barrier()

    cp_i[0].wait()
    cp_g[0] = pltpu.async_copy(shared.at[idx_v0], rows_v0, sem_g0)

    for ch in range(nchunks):
        b = ch & 1
        nb = 1 - b
        if ch + 1 < nchunks:
            if cp_o[nb] is not None:
                cp_o[nb].wait()
            cp_i[ch + 1].wait()
            cp_g[nb] = pltpu.async_copy(shared.at[idx_v[nb]], rows_v[nb],
                                        sem_g[nb])
        cp_g[b].wait()
        if ch + 2 < nchunks:
            off = base + (ch + 2) * chunk
            cp_i[ch + 2] = pltpu.async_copy(idx_hbm.at[pl.ds(off, chunk)],
                                            idx_v[b], sem_i[b])
        cp_o[b] = pltpu.async_copy(
            rows_v[b], out_hbm.at[pl.ds(base + ch * chunk, chunk)], sem_o[b])
    for cp in cp_o:
        if cp is not None:
            cp.wait()


def kernel(idx, alpha):
    batch, hist = idx.shape
    n = alpha.shape[0]

    n_pad = (-n) % 1024
    ap = jnp.pad(alpha, (0, n_pad), constant_values=-jnp.inf)
    table = _softmax_table(ap.reshape(-1, 128)).reshape(-1)

    bflat = batch * hist
    assert bflat % (8 * _NW) == 0
    b_per_w = bflat // _NW
    chunk = b_per_w
    nchunks = 1
    while chunk * 16 > 208 * 1024 or chunk % _L != 0:
        nchunks += 1
        while b_per_w % nchunks != 0:
            nchunks += 1
        chunk = b_per_w // nchunks

    n_table = n + n_pad
    assert n_table % (8 * _NS) == 0
    t_per_t = n_table // _NS

    mesh = plsc.VectorSubcoreMesh(core_axis_name="c", subcore_axis_name="s")
    gather = pl.kernel(
        functools.partial(_sc_gather_body, nchunks, chunk, b_per_w, t_per_t),
        out_type=jax.ShapeDtypeStruct((bflat,), jnp.float32),
        mesh=mesh,
        scratch_types=[
            pltpu.VMEM((chunk,), jnp.int32),
            pltpu.VMEM((chunk,), jnp.int32),
            pltpu.VMEM((chunk,), jnp.float32),
            pltpu.VMEM((chunk,), jnp.float32),
            pltpu.VMEM_SHARED((n_table,), jnp.float32),
            pltpu.SemaphoreType.DMA,
            pltpu.SemaphoreType.DMA,
            pltpu.SemaphoreType.DMA,
            pltpu.SemaphoreType.DMA,
            pltpu.SemaphoreType.DMA,
            pltpu.SemaphoreType.DMA,
        ],
    )
    out_flat = gather(table, idx.reshape(-1))
    return out_flat.reshape(batch, hist)


# speedup vs baseline: 1.0164x; 1.0164x over previous
import functools

import jax
import jax.numpy as jnp
from jax import lax
from jax.experimental import pallas as pl
from jax.experimental.pallas import tpu as pltpu
from jax.experimental.pallas import tpu_sc as plsc

_NC = 2
_NS = 16
_NW = _NC * _NS
_L = 16


def _softmax_body(alpha_ref, out_ref):
    a = alpha_ref[...]
    m = jnp.max(a)
    e = jnp.exp(a - m)
    out_ref[...] = e / jnp.sum(e)


def _softmax_table(alpha_padded_2d):
    return pl.pallas_call(
        _softmax_body,
        out_shape=jax.ShapeDtypeStruct(alpha_padded_2d.shape, jnp.float32),
    )(alpha_padded_2d)


def _sc_gather_body(nchunks, chunk, b_per_w, t_per_t, table_hbm, idx_hbm,
                    out_hbm, idx_v0, idx_v1, rows_v0, rows_v1, shared,
                    sem_i0, sem_i1, sem_g0, sem_g1, sem_o0, sem_o1):
    sid = lax.axis_index("s")
    wid = sid * _NC + lax.axis_index("c")
    base = wid * b_per_w
    idx_v = [idx_v0, idx_v1]
    rows_v = [rows_v0, rows_v1]
    sem_i = [sem_i0, sem_i1]
    sem_g = [sem_g0, sem_g1]
    sem_o = [sem_o0, sem_o1]

    cp_i = [None] * nchunks
    cp_g = [None, None]
    cp_o = [None, None]
    cp_i[0] = pltpu.async_copy(idx_hbm.at[pl.ds(base, chunk)], idx_v0, sem_i0)
    if nchunks > 1:
        cp_i[1] = pltpu.async_copy(idx_hbm.at[pl.ds(base + chunk, chunk)],
                                   idx_v1, sem_i1)

    toff = sid * t_per_t
    pieces = []
    done = 0
    while done < t_per_t:
        piece = min(chunk, t_per_t - done)
        pieces.append((done, piece))
        done += piece
    bufs = [rows_v0, rows_v1]
    h1 = [None, None]
    h2 = [None, None]
    d0, p0 = pieces[0]
    h1[0] = pltpu.async_copy(table_hbm.at[pl.ds(toff + d0, p0)],
                             rows_v0.at[pl.ds(0, p0)], sem_g0)
    for p, (doff, plen) in enumerate(pieces):
        b = p & 1
        h1[b].wait()
        h2[b] = pltpu.async_copy(bufs[b].at[pl.ds(0, plen)],
                                 shared.at[pl.ds(toff + doff, plen)],
                                 sem_o[b])
        if p + 1 < len(pieces):
            nd, npc = pieces[p + 1]
            if h2[1 - b] is not None:
                h2[1 - b].wait()
            h1[1 - b] = pltpu.async_copy(
                table_hbm.at[pl.ds(toff + nd, npc)],
                bufs[1 - b].at[pl.ds(0, npc)], sem_g0)
    for cp in h2:
        if cp is not None:
            cp.wait()
    plsc.subcore_barrier()

    cp_i[0].wait()
    cp_g[0] = pltpu.async_copy(shared.at[idx_v0], rows_v0, sem_g0)

    for ch in range(nchunks):
        b = ch & 1
        nb = 1 - b
        if ch + 1 < nchunks:
            if cp_o[nb] is not None:
                cp_o[nb].wait()
            cp_i[ch + 1].wait()
            cp_g[nb] = pltpu.async_copy(shared.at[idx_v[nb]], rows_v[nb],
                                        sem_g[nb])
        cp_g[b].wait()
        if ch + 2 < nchunks:
            off = base + (ch + 2) * chunk
            cp_i[ch + 2] = pltpu.async_copy(idx_hbm.at[pl.ds(off, chunk)],
                                            idx_v[b], sem_i[b])
        cp_o[b] = pltpu.async_copy(
            rows_v[b], out_hbm.at[pl.ds(base + ch * chunk, chunk)], sem_o[b])
    for cp in cp_o:
        if cp is not None:
            cp.wait()


def kernel(idx, alpha):
    batch, hist = idx.shape
    n = alpha.shape[0]

    n_pad = (-n) % 1024
    ap = jnp.pad(alpha, (0, n_pad), constant_values=-jnp.inf)
    table = _softmax_table(ap.reshape(-1, 128)).reshape(-1)

    bflat = batch * hist
    assert bflat % (8 * _NW) == 0
    b_per_w = bflat // _NW
    chunk = b_per_w
    nchunks = 1
    while chunk * 16 > 208 * 1024 or chunk % _L != 0:
        nchunks += 1
        while b_per_w % nchunks != 0:
            nchunks += 1
        chunk = b_per_w // nchunks

    n_table = n + n_pad
    assert n_table % (8 * _NS) == 0
    t_per_t = n_table // _NS

    mesh = plsc.VectorSubcoreMesh(core_axis_name="c", subcore_axis_name="s")
    gather = pl.kernel(
        functools.partial(_sc_gather_body, nchunks, chunk, b_per_w, t_per_t),
        out_type=jax.ShapeDtypeStruct((bflat,), jnp.float32),
        mesh=mesh,
        scratch_types=[
            pltpu.VMEM((chunk,), jnp.int32),
            pltpu.VMEM((chunk,), jnp.int32),
            pltpu.VMEM((chunk,), jnp.float32),
            pltpu.VMEM((chunk,), jnp.float32),
            pltpu.VMEM_SHARED((n_table,), jnp.float32),
            pltpu.SemaphoreType.DMA,
            pltpu.SemaphoreType.DMA,
            pltpu.SemaphoreType.DMA,
            pltpu.SemaphoreType.DMA,
            pltpu.SemaphoreType.DMA,
            pltpu.SemaphoreType.DMA,
        ],
    )
    out_flat = gather(table, idx.reshape(-1))
    return out_flat.reshape(batch, hist)
